# baseline (speedup 1.0000x reference)
import functools

import jax
import jax.numpy as jnp
from jax import lax
from jax.experimental import pallas as pl
from jax.experimental.pallas import tpu as pltpu
from jax.experimental.pallas import tpu_sc as plsc

_EMB_DIM = 64
_BATCH = 16384
_NC = 2
_NS = 16
_NW = _NC * _NS
_B_PER_W = _BATCH // _NW
_L = 16
_NG = _B_PER_W // _L
_NBUF = 4

_mesh = plsc.VectorSubcoreMesh(core_axis_name="c", subcore_axis_name="s")


@functools.partial(
    pl.kernel,
    mesh=_mesh,
    out_type=jax.ShapeDtypeStruct((_BATCH * _EMB_DIM,), jnp.float32),
    scratch_types=[
        pltpu.VMEM((_B_PER_W,), jnp.int32),
        pltpu.VMEM((_NBUF * _L * 8, _EMB_DIM), jnp.float32),
        pltpu.VMEM((_B_PER_W * _EMB_DIM,), jnp.float32),
        pltpu.SemaphoreType.DMA,
    ],
    compiler_params=pltpu.CompilerParams(needs_layout_passes=False),
)
def _gather_kernel(idx_hbm, table_hbm, out_hbm, idx_v, tiles_v, out_v, sem):
    wid = lax.axis_index("s") * _NC + lax.axis_index("c")
    base = wid * _B_PER_W
    pltpu.sync_copy(idx_hbm.at[pl.ds(base, _B_PER_W)], idx_v)
    lane = lax.iota(jnp.int32, 16)

    def fire_group(g, slot):
        idxg = idx_v[pl.ds(g * _L, _L)]
        t_vec = lax.bitwise_and(idxg, ~7)
        for j in range(_L):
            t_s = pl.multiple_of(jnp.max(jnp.where(lane == j, t_vec, 0)), 8)
            pltpu.async_copy(
                table_hbm.at[pl.ds(t_s, 8)],
                tiles_v.at[pl.ds(slot * (_L * 8) + j * 8, 8)],
                sem,
            )

    def extract_group(g, slot):
        idxg = idx_v[pl.ds(g * _L, _L)]
        r_vec = lax.bitwise_and(idxg, 7)
        for j in range(_L):
            r_s = jnp.max(jnp.where(lane == j, r_vec, 0))
            rowbase = slot * (_L * 8) + j * 8 + r_s
            pos = (g * _L + j) * _EMB_DIM
            for c in range(_EMB_DIM // _L):
                out_v[pl.ds(pos + c * _L, _L)] = tiles_v[
                    rowbase, pl.ds(c * _L, _L)
                ]

    for g in range(_NBUF - 1):
        fire_group(g, g)

    def body(g, carry):
        slot = lax.rem(g, _NBUF)

        @pl.when(g + (_NBUF - 1) < _NG)
        def _fire_ahead():
            fire_group(g + (_NBUF - 1), lax.rem(g + (_NBUF - 1), _NBUF))

        pltpu.make_async_copy(
            table_hbm.at[pl.ds(0, _L * 8)],
            tiles_v.at[pl.ds(slot * (_L * 8), _L * 8)],
            sem,
        ).wait()
        extract_group(g, slot)
        return carry

    lax.fori_loop(0, _NG, body, 0)
    pltpu.sync_copy(out_v, out_hbm.at[pl.ds(base * _EMB_DIM, _B_PER_W * _EMB_DIM)])


def kernel(input, table):
    out = _gather_kernel(input.astype(jnp.int32), table)
    return out.reshape(1, 1, -1)


# speedup vs baseline: 1.6016x; 1.6016x over previous
import functools

import jax
import jax.numpy as jnp
from jax import lax
from jax.experimental import pallas as pl
from jax.experimental.pallas import tpu as pltpu
from jax.experimental.pallas import tpu_sc as plsc

_EMB_DIM = 64
_BATCH = 16384
_NC = 2
_NS = 16
_NW = _NC * _NS
_B_PER_W = _BATCH // _NW
_L = 16
_NG = _B_PER_W // _L

_mesh = plsc.VectorSubcoreMesh(core_axis_name="c", subcore_axis_name="s")


@functools.partial(
    pl.kernel,
    mesh=_mesh,
    out_type=jax.ShapeDtypeStruct((_BATCH * _EMB_DIM,), jnp.float32),
    scratch_types=[
        pltpu.VMEM((_B_PER_W,), jnp.int32),
        pltpu.VMEM((_B_PER_W * _EMB_DIM,), jnp.float32),
        pltpu.SemaphoreType.DMA,
    ],
    compiler_params=pltpu.CompilerParams(needs_layout_passes=False),
)
def _gather_kernel(idx_hbm, table_hbm, out_hbm, idx_v, out_v, sem):
    wid = lax.axis_index("s") * _NC + lax.axis_index("c")
    base = wid * _B_PER_W
    pltpu.sync_copy(idx_hbm.at[pl.ds(base, _B_PER_W)], idx_v)
    lane = lax.iota(jnp.int32, 16)

    def body(g, carry):
        idxg = idx_v[pl.ds(g * _L, _L)]
        for j in range(_L):
            row_s = jnp.max(jnp.where(lane == j, idxg, 0))
            pltpu.async_copy(
                table_hbm.at[row_s, 0],
                out_v.at[pl.ds((g * _L + j) * _EMB_DIM, _EMB_DIM)],
                sem,
            )
        return carry

    lax.fori_loop(0, _NG, body, 0)
    pltpu.make_async_copy(
        out_hbm.at[pl.ds(0, _B_PER_W * _EMB_DIM)], out_v, sem
    ).wait()
    pltpu.sync_copy(out_v, out_hbm.at[pl.ds(base * _EMB_DIM, _B_PER_W * _EMB_DIM)])


def kernel(input, table):
    table3 = table.reshape(1000000, 1, _EMB_DIM)
    out = _gather_kernel(input.astype(jnp.int32), table3)
    return out.reshape(1, 1, -1)
